# baseline (speedup 1.0000x reference)
import functools

import jax
import jax.numpy as jnp
from jax import lax
from jax.experimental import pallas as pl
from jax.experimental.pallas import tpu as pltpu, tpu_sc as plsc

N = 10000
E = 320000
NPR = 100000
D_IN = 128
H0 = 512
H1 = 256
H2 = 128
H4 = 256
OUT = 128
CAT = 4

NP_ = 10240
NSUB = 16
E_PAD = 327680
CH1 = 160
CH3 = 80
CHP = 27
PR_PAD = 110592

_mesh = plsc.VectorSubcoreMesh(core_axis_name="c", subcore_axis_name="s")


NACC = 10112


@functools.partial(
    pl.kernel,
    out_type=jax.ShapeDtypeStruct((2 * NACC, 128), jnp.float32),
    mesh=_mesh,
    scratch_types=[
        pltpu.VMEM_SHARED((NACC, 128), jnp.float32),
        pltpu.VMEM((CH1, 128), jnp.int32),
        pltpu.VMEM((128, 128), jnp.float32),
    ],
)
def _deg_kernel(cat_hbm, zeros_hbm, ones_hbm, out_hbm, acc, idx_v, ones_v):
    c = lax.axis_index("c")
    s = lax.axis_index("s")
    pltpu.sync_copy(ones_hbm, ones_v)
    pltpu.sync_copy(zeros_hbm, acc.at[pl.ds(s * 632, 632)])
    plsc.subcore_barrier()
    pltpu.sync_copy(cat_hbm.at[pl.ds(c * 2560 + s * CH1, CH1)], idx_v)

    def scat(j, _):
        pltpu.sync_copy(ones_v, acc.at[idx_v.at[j]], add=True)
        return 0
    lax.fori_loop(0, CH1, scat, 0)

    plsc.subcore_barrier()
    pltpu.sync_copy(acc.at[pl.ds(s * 632, 632)],
                    out_hbm.at[pl.ds(c * NACC + s * 632, 632)])


def _gsc_loop(v_hbm, sidx, didx, acc, buf0, buf1, sem, nchunks):
    pltpu.async_copy(v_hbm.at[sidx.at[0]], buf0, sem)

    def body2(i, _):
        j0 = 2 * i
        j1 = j0 + 1
        pltpu.async_copy(v_hbm.at[sidx.at[j1]], buf1, sem)
        pltpu.make_async_copy(v_hbm.at[sidx.at[j0]], buf0, sem).wait()
        pltpu.sync_copy(buf0, acc.at[didx.at[j0]], add=True)
        pltpu.async_copy(v_hbm.at[sidx.at[j1 + 1]], buf0, sem)
        pltpu.make_async_copy(v_hbm.at[sidx.at[j1]], buf1, sem).wait()
        pltpu.sync_copy(buf1, acc.at[didx.at[j1]], add=True)
        return 0
    lax.fori_loop(0, nchunks // 2, body2, 0)
    pltpu.make_async_copy(v_hbm.at[sidx.at[nchunks]], buf0, sem).wait()


CW = 128
CHW = 80


@functools.partial(
    pl.kernel,
    out_type=jax.ShapeDtypeStruct((2 * NACC, 128), jnp.float32),
    mesh=_mesh,
    scratch_types=[
        pltpu.VMEM_SHARED((NACC, 128), jnp.float32),
        pltpu.VMEM((CHW + 8, CW), jnp.int32),
        pltpu.VMEM((CHW // 2, CW), jnp.int32),
        pltpu.VMEM((CW, 128), jnp.float32),
        pltpu.VMEM((CW, 128), jnp.float32),
        pltpu.SemaphoreType.DMA,
    ],
)
def _agg1_kernel(src_hbm, dst_hbm, v_hbm, zeros_hbm, out_hbm,
                 acc, sidx, didx, buf0, buf1, sem):
    c = lax.axis_index("c")
    s = lax.axis_index("s")
    w = c * NSUB + s
    hh = CHW // 2
    pltpu.sync_copy(zeros_hbm, acc.at[pl.ds(s * 632, 632)])
    pltpu.sync_copy(src_hbm.at[pl.ds(w * (CHW + 8), CHW + 8)], sidx)
    plsc.subcore_barrier()

    pltpu.async_copy(v_hbm.at[sidx.at[0]], buf0, sem)
    for h in (0, 1):
        pltpu.sync_copy(dst_hbm.at[pl.ds(w * CHW + h * hh, hh)], didx)

        def body2(i, _, h=h):
            j0 = h * hh + 2 * i
            j1 = j0 + 1
            d0 = 2 * i
            d1 = d0 + 1
            pltpu.async_copy(v_hbm.at[sidx.at[j1]], buf1, sem)
            pltpu.make_async_copy(v_hbm.at[sidx.at[j0]], buf0, sem).wait()
            pltpu.sync_copy(buf0, acc.at[didx.at[d0]], add=True)
            pltpu.async_copy(v_hbm.at[sidx.at[j1 + 1]], buf0, sem)
            pltpu.make_async_copy(v_hbm.at[sidx.at[j1]], buf1, sem).wait()
            pltpu.sync_copy(buf1, acc.at[didx.at[d1]], add=True)
            return 0
        lax.fori_loop(0, hh // 2, body2, 0)
    pltpu.make_async_copy(v_hbm.at[sidx.at[CHW]], buf0, sem).wait()
    plsc.subcore_barrier()
    pltpu.sync_copy(acc.at[pl.ds(s * 632, 632)],
                    out_hbm.at[pl.ds(c * NACC + s * 632, 632)])


@functools.partial(
    pl.kernel,
    out_type=[jax.ShapeDtypeStruct((PR_PAD, 128), jnp.float32)
              for _ in range(8)],
    mesh=_mesh,
    scratch_types=[
        pltpu.VMEM((CHP + 5, 128), jnp.int32),
        pltpu.VMEM((CHP + 5, 128), jnp.int32),
        pltpu.VMEM((128, 128), jnp.float32),
        pltpu.VMEM((128, 128), jnp.float32),
        pltpu.VMEM((128, 128), jnp.float32),
        pltpu.VMEM((128, 128), jnp.float32),
        pltpu.SemaphoreType.DMA,
        pltpu.SemaphoreType.DMA,
    ],
)
def _pairs_kernel(m_hbm, ls_hbm, pa_hbm, pb_hbm, na_hbm, nb_hbm,
                  pma_hbm, pmb_hbm, plsa_hbm, plsb_hbm,
                  nma_hbm, nmb_hbm, nlsa_hbm, nlsb_hbm,
                  ia, ib, bufa0, bufb0, bufa1, bufb1, sem, semw):
    c = lax.axis_index("c")
    s = lax.axis_index("s")
    w = c * NSUB + s

    def phase(v_hbm, outa_hbm, outb_hbm):
        row = lambda j: pl.ds((w * CHP + j) * 128, 128)

        def fire_g(j, ba, bb):
            pltpu.async_copy(v_hbm.at[ia.at[j]], ba, sem)
            pltpu.async_copy(v_hbm.at[ib.at[j]], bb, sem)

        def retire(j, ba, bb):
            pltpu.make_async_copy(v_hbm.at[ia.at[j]], ba, sem).wait()
            pltpu.async_copy(ba, outa_hbm.at[row(j)], semw)
            pltpu.make_async_copy(v_hbm.at[ib.at[j]], bb, sem).wait()
            pltpu.async_copy(bb, outb_hbm.at[row(j)], semw)

        def drain_w(j, ba, bb):
            pltpu.make_async_copy(ba, outa_hbm.at[row(j)], semw).wait()
            pltpu.make_async_copy(bb, outb_hbm.at[row(j)], semw).wait()

        fire_g(0, bufa0, bufb0)
        fire_g(1, bufa1, bufb1)
        retire(0, bufa0, bufb0)
        retire(1, bufa1, bufb1)

        def body2(i, _):
            j0 = 2 + 2 * i
            j1 = j0 + 1
            drain_w(j0 - 2, bufa0, bufb0)
            fire_g(j0, bufa0, bufb0)
            drain_w(j1 - 2, bufa1, bufb1)
            fire_g(j1, bufa1, bufb1)
            retire(j0, bufa0, bufb0)
            retire(j1, bufa1, bufb1)
            return 0
        lax.fori_loop(0, (CHP - 3) // 2, body2, 0)
        drain_w(CHP - 3, bufa0, bufb0)
        fire_g(CHP - 1, bufa0, bufb0)
        retire(CHP - 1, bufa0, bufb0)
        drain_w(CHP - 2, bufa1, bufb1)
        drain_w(CHP - 1, bufa0, bufb0)

    pltpu.sync_copy(pa_hbm.at[pl.ds(w * (CHP + 5), CHP + 5)], ia)
    pltpu.sync_copy(pb_hbm.at[pl.ds(w * (CHP + 5), CHP + 5)], ib)
    phase(m_hbm, pma_hbm, pmb_hbm)
    phase(ls_hbm, plsa_hbm, plsb_hbm)
    pltpu.sync_copy(na_hbm.at[pl.ds(w * (CHP + 5), CHP + 5)], ia)
    pltpu.sync_copy(nb_hbm.at[pl.ds(w * (CHP + 5), CHP + 5)], ib)
    phase(m_hbm, nma_hbm, nmb_hbm)
    phase(ls_hbm, nlsa_hbm, nlsb_hbm)


ROWS_BLK = 160


def _head_body(ma_ref, mb_ref, lsa_ref, lsb_ref, noise_ref,
               wd1_ref, bd1_ref, wdx_ref, bdx_ref,
               wc1_ref, bc1_ref, wca_ref, bca_ref,
               x_ref, a_ref, mean_ref, ls_ref):
    mean = ma_ref[...] + mb_ref[...]
    ls = lsa_ref[...] + lsb_ref[...]
    mean_ref[...] = mean
    ls_ref[...] = ls
    noise = noise_ref[...]
    z = noise * jnp.exp(ls) + mean
    h = jnp.maximum(jnp.dot(z, wd1_ref[...], preferred_element_type=jnp.float32)
                    + bd1_ref[...], 0.0)
    x_ref[...] = jnp.maximum(
        jnp.dot(h, wdx_ref[...], preferred_element_type=jnp.float32) + bdx_ref[...], 0.0)
    c = jnp.maximum(jnp.dot(z, wc1_ref[...], preferred_element_type=jnp.float32)
                    + bc1_ref[...], 0.0)
    logits = jnp.dot(c, wca_ref[...], preferred_element_type=jnp.float32) + bca_ref[...]
    col = jax.lax.broadcasted_iota(jnp.int32, logits.shape, 1)
    valid = col < CAT
    logits = jnp.where(valid, logits, -jnp.inf)
    m = jnp.max(logits, axis=-1, keepdims=True)
    e = jnp.where(valid, jnp.exp(logits - m), 0.0)
    a_ref[...] = e / jnp.sum(e, axis=-1, keepdims=True)


def _head(ma, mb, lsa, lsb, noise, Wd1, bd1, WdX, bdX, Wc1, bc1, WcA_p, bcA_p):
    grid = NPR // ROWS_BLK
    row_spec = pl.BlockSpec((ROWS_BLK, H2), lambda i: (i, 0))
    full = lambda a: pl.BlockSpec(a.shape, lambda i: tuple(0 for _ in a.shape))
    args = (ma, mb, lsa, lsb, noise, Wd1, bd1, WdX, bdX, Wc1, bc1, WcA_p, bcA_p)
    out_x, out_a, out_mean, out_ls = pl.pallas_call(
        _head_body,
        grid=(grid,),
        in_specs=[row_spec] * 5 + [full(a) for a in args[5:]],
        out_specs=[pl.BlockSpec((ROWS_BLK, OUT), lambda i: (i, 0)),
                   pl.BlockSpec((ROWS_BLK, 128), lambda i: (i, 0)),
                   row_spec, row_spec],
        out_shape=[jax.ShapeDtypeStruct((NPR, OUT), jnp.float32),
                   jax.ShapeDtypeStruct((NPR, 128), jnp.float32),
                   jax.ShapeDtypeStruct((NPR, H2), jnp.float32),
                   jax.ShapeDtypeStruct((NPR, H2), jnp.float32)],
    )(*args)
    return out_x, out_a, out_mean, out_ls


def kernel(x, edge_index, pos_edge_index, neg_edge_index, temp, W0, b0, W1, b1,
           Wm, bm, Wls, bls, Wd1, bd1, WdX, bdX, Wc1, bc1, WcA, bcA):
    f32 = jnp.float32
    epad = jnp.full((E_PAD - E,), NACC - 1, jnp.int32)
    src_flat = jnp.concatenate([edge_index[0], epad])
    dst_flat = jnp.concatenate([edge_index[1], epad])

    def _with_pad_rows(flat, width, nchunks, npad):
        a = flat.reshape(32, nchunks, width)
        pr = jnp.zeros((32, npad, width), jnp.int32)
        return jnp.concatenate([a, pr], axis=1).reshape(-1, width)

    srcP = _with_pad_rows(src_flat, CW, CHW, 8)
    dstP = dst_flat.reshape(-1, CW)
    deg_cat = jnp.concatenate([src_flat.reshape(-1, 128),
                               dst_flat.reshape(-1, 128)])
    ppad = jnp.zeros((CHP * 128 * 32 - NPR,), jnp.int32)
    pa = _with_pad_rows(jnp.concatenate([pos_edge_index[0], ppad]), 128, CHP, 5)
    pb = _with_pad_rows(jnp.concatenate([pos_edge_index[1], ppad]), 128, CHP, 5)
    na = _with_pad_rows(jnp.concatenate([neg_edge_index[0], ppad]), 128, CHP, 5)
    nb = _with_pad_rows(jnp.concatenate([neg_edge_index[1], ppad]), 128, CHP, 5)
    xp = jnp.zeros((NACC, D_IN), f32).at[:N].set(x)
    zeros128 = jnp.zeros((632, 128), f32)
    ones128 = jnp.ones((128, 128), f32)


    cnt = _deg_kernel(deg_cat, zeros128, ones128)
    ns = lax.rsqrt(jnp.clip(cnt[:NACC, 0], 1.0))
    nd = lax.rsqrt(jnp.clip(cnt[NACC:, 0], 1.0))

    v1 = xp * ns[:, None]
    aggp = _agg1_kernel(srcP, dstP, v1, zeros128)
    agg1 = aggp[:NACC] + aggp[NACC:]
    h1 = jax.nn.relu((agg1 * nd[:, None]) @ W0 + b0)

    t = (h1 * ns[:, None]) @ W1
    a2h0 = _agg1_kernel(srcP, dstP, t[:, :128], zeros128)
    a2h1 = _agg1_kernel(srcP, dstP, t[:, 128:], zeros128)
    agg2 = jnp.concatenate([a2h0[:NACC] + a2h0[NACC:],
                            a2h1[:NACC] + a2h1[NACC:]], axis=1)
    h2 = jax.nn.relu(agg2 * nd[:, None] + b1)

    M = h2 @ Wm + 0.5 * bm
    LS = h2 @ Wls + 0.5 * bls
    (pma, pmb, plsa, plsb,
     nma, nmb, nlsa, nlsb) = _pairs_kernel(M, LS, pa, pb, na, nb)

    kp, kn = jax.random.split(jax.random.key(42))
    noise_p = jax.random.normal(kp, (NPR, H2), f32)
    noise_n = jax.random.normal(kn, (NPR, H2), f32)

    WcA_p = jnp.zeros((H2 // 2, 128), f32).at[:, :CAT].set(WcA)
    bcA_p = jnp.zeros((1, 128), f32).at[0, :CAT].set(bcA)
    head = functools.partial(_head, Wd1=Wd1, bd1=bd1.reshape(1, -1), WdX=WdX,
                             bdX=bdX.reshape(1, -1), Wc1=Wc1,
                             bc1=bc1.reshape(1, -1), WcA_p=WcA_p, bcA_p=bcA_p)
    posX, posA, pos_mean, pos_ls = head(pma, pmb, plsa, plsb, noise_p)
    negX, negA, neg_mean, neg_ls = head(nma, nmb, nlsa, nlsb, noise_n)
    return (posA[:, :CAT], negA[:, :CAT], posX, negX,
            pos_mean, neg_mean, pos_ls, neg_ls)


# speedup vs baseline: 1.1291x; 1.1291x over previous
import functools

import jax
import jax.numpy as jnp
from jax import lax
from jax.experimental import pallas as pl
from jax.experimental.pallas import tpu as pltpu, tpu_sc as plsc

N = 10000
E = 320000
NPR = 100000
D_IN = 128
H0 = 512
H1 = 256
H2 = 128
H4 = 256
OUT = 128
CAT = 4

NP_ = 10240
NSUB = 16
E_PAD = 327680
CH1 = 160
CH3 = 80
CHP = 27
PR_PAD = 110592

_mesh = plsc.VectorSubcoreMesh(core_axis_name="c", subcore_axis_name="s")


NACC = 10112


@functools.partial(
    pl.kernel,
    out_type=jax.ShapeDtypeStruct((2 * NACC, 128), jnp.float32),
    mesh=_mesh,
    scratch_types=[
        pltpu.VMEM_SHARED((NACC, 128), jnp.float32),
        pltpu.VMEM((CH1, 128), jnp.int32),
        pltpu.VMEM((128, 128), jnp.float32),
    ],
)
def _deg_kernel(cat_hbm, zeros_hbm, ones_hbm, out_hbm, acc, idx_v, ones_v):
    c = lax.axis_index("c")
    s = lax.axis_index("s")
    pltpu.sync_copy(ones_hbm, ones_v)
    pltpu.sync_copy(zeros_hbm, acc.at[pl.ds(s * 632, 632)])
    plsc.subcore_barrier()
    pltpu.sync_copy(cat_hbm.at[pl.ds(c * 2560 + s * CH1, CH1)], idx_v)

    def scat(j, _):
        pltpu.sync_copy(ones_v, acc.at[idx_v.at[j]], add=True)
        return 0
    lax.fori_loop(0, CH1, scat, 0)

    plsc.subcore_barrier()
    pltpu.sync_copy(acc.at[pl.ds(s * 632, 632)],
                    out_hbm.at[pl.ds(c * NACC + s * 632, 632)])


def _gsc_loop(v_hbm, sidx, didx, acc, buf0, buf1, sem, nchunks):
    pltpu.async_copy(v_hbm.at[sidx.at[0]], buf0, sem)

    def body2(i, _):
        j0 = 2 * i
        j1 = j0 + 1
        pltpu.async_copy(v_hbm.at[sidx.at[j1]], buf1, sem)
        pltpu.make_async_copy(v_hbm.at[sidx.at[j0]], buf0, sem).wait()
        pltpu.sync_copy(buf0, acc.at[didx.at[j0]], add=True)
        pltpu.async_copy(v_hbm.at[sidx.at[j1 + 1]], buf0, sem)
        pltpu.make_async_copy(v_hbm.at[sidx.at[j1]], buf1, sem).wait()
        pltpu.sync_copy(buf1, acc.at[didx.at[j1]], add=True)
        return 0
    lax.fori_loop(0, nchunks // 2, body2, 0)
    pltpu.make_async_copy(v_hbm.at[sidx.at[nchunks]], buf0, sem).wait()


CW = 128
CHW = 80


@functools.partial(
    pl.kernel,
    out_type=jax.ShapeDtypeStruct((2 * NACC, 128), jnp.float32),
    mesh=_mesh,
    scratch_types=[
        pltpu.VMEM_SHARED((NACC, 128), jnp.float32),
        pltpu.VMEM((CHW + 8, CW), jnp.int32),
        pltpu.VMEM((CHW // 2, CW), jnp.int32),
        pltpu.VMEM((CW, 128), jnp.float32),
        pltpu.VMEM((CW, 128), jnp.float32),
        pltpu.SemaphoreType.DMA,
    ],
)
def _agg1_kernel(src_hbm, dst_hbm, v_hbm, zeros_hbm, out_hbm,
                 acc, sidx, didx, buf0, buf1, sem):
    c = lax.axis_index("c")
    s = lax.axis_index("s")
    w = c * NSUB + s
    hh = CHW // 2
    pltpu.sync_copy(zeros_hbm, acc.at[pl.ds(s * 632, 632)])
    pltpu.sync_copy(src_hbm.at[pl.ds(w * (CHW + 8), CHW + 8)], sidx)
    plsc.subcore_barrier()

    pltpu.async_copy(v_hbm.at[sidx.at[0]], buf0, sem)
    for h in (0, 1):
        pltpu.sync_copy(dst_hbm.at[pl.ds(w * CHW + h * hh, hh)], didx)

        def body2(i, _, h=h):
            j0 = h * hh + 2 * i
            j1 = j0 + 1
            d0 = 2 * i
            d1 = d0 + 1
            pltpu.async_copy(v_hbm.at[sidx.at[j1]], buf1, sem)
            pltpu.make_async_copy(v_hbm.at[sidx.at[j0]], buf0, sem).wait()
            pltpu.sync_copy(buf0, acc.at[didx.at[d0]], add=True)
            pltpu.async_copy(v_hbm.at[sidx.at[j1 + 1]], buf0, sem)
            pltpu.make_async_copy(v_hbm.at[sidx.at[j1]], buf1, sem).wait()
            pltpu.sync_copy(buf1, acc.at[didx.at[d1]], add=True)
            return 0
        lax.fori_loop(0, hh // 2, body2, 0)
    pltpu.make_async_copy(v_hbm.at[sidx.at[CHW]], buf0, sem).wait()
    plsc.subcore_barrier()
    pltpu.sync_copy(acc.at[pl.ds(s * 632, 632)],
                    out_hbm.at[pl.ds(c * NACC + s * 632, 632)])


@functools.partial(
    pl.kernel,
    out_type=[jax.ShapeDtypeStruct((PR_PAD, 128), jnp.float32)
              for _ in range(8)],
    mesh=_mesh,
    scratch_types=[
        pltpu.VMEM((CHP + 5, 128), jnp.int32),
        pltpu.VMEM((CHP + 5, 128), jnp.int32),
        pltpu.VMEM((128, 128), jnp.float32),
        pltpu.VMEM((128, 128), jnp.float32),
        pltpu.VMEM((128, 128), jnp.float32),
        pltpu.VMEM((128, 128), jnp.float32),
        pltpu.SemaphoreType.DMA,
        pltpu.SemaphoreType.DMA,
    ],
)
def _pairs_kernel(m_hbm, ls_hbm, pa_hbm, pb_hbm, na_hbm, nb_hbm,
                  pma_hbm, pmb_hbm, plsa_hbm, plsb_hbm,
                  nma_hbm, nmb_hbm, nlsa_hbm, nlsb_hbm,
                  ia, ib, bufa0, bufb0, bufa1, bufb1, sem, semw):
    c = lax.axis_index("c")
    s = lax.axis_index("s")
    w = c * NSUB + s

    def phase(v_hbm, outa_hbm, outb_hbm):
        row = lambda j: pl.ds((w * CHP + j) * 128, 128)

        def fire_g(j, ba, bb):
            pltpu.async_copy(v_hbm.at[ia.at[j]], ba, sem)
            pltpu.async_copy(v_hbm.at[ib.at[j]], bb, sem)

        def retire(j, ba, bb):
            pltpu.make_async_copy(v_hbm.at[ia.at[j]], ba, sem).wait()
            pltpu.async_copy(ba, outa_hbm.at[row(j)], semw)
            pltpu.make_async_copy(v_hbm.at[ib.at[j]], bb, sem).wait()
            pltpu.async_copy(bb, outb_hbm.at[row(j)], semw)

        def drain_w(j, ba, bb):
            pltpu.make_async_copy(ba, outa_hbm.at[row(j)], semw).wait()
            pltpu.make_async_copy(bb, outb_hbm.at[row(j)], semw).wait()

        fire_g(0, bufa0, bufb0)
        fire_g(1, bufa1, bufb1)
        retire(0, bufa0, bufb0)
        retire(1, bufa1, bufb1)

        def body2(i, _):
            j0 = 2 + 2 * i
            j1 = j0 + 1
            drain_w(j0 - 2, bufa0, bufb0)
            fire_g(j0, bufa0, bufb0)
            drain_w(j1 - 2, bufa1, bufb1)
            fire_g(j1, bufa1, bufb1)
            retire(j0, bufa0, bufb0)
            retire(j1, bufa1, bufb1)
            return 0
        lax.fori_loop(0, (CHP - 3) // 2, body2, 0)
        drain_w(CHP - 3, bufa0, bufb0)
        fire_g(CHP - 1, bufa0, bufb0)
        retire(CHP - 1, bufa0, bufb0)
        drain_w(CHP - 2, bufa1, bufb1)
        drain_w(CHP - 1, bufa0, bufb0)

    pltpu.sync_copy(pa_hbm.at[pl.ds(w * (CHP + 5), CHP + 5)], ia)
    pltpu.sync_copy(pb_hbm.at[pl.ds(w * (CHP + 5), CHP + 5)], ib)
    phase(m_hbm, pma_hbm, pmb_hbm)
    phase(ls_hbm, plsa_hbm, plsb_hbm)
    pltpu.sync_copy(na_hbm.at[pl.ds(w * (CHP + 5), CHP + 5)], ia)
    pltpu.sync_copy(nb_hbm.at[pl.ds(w * (CHP + 5), CHP + 5)], ib)
    phase(m_hbm, nma_hbm, nmb_hbm)
    phase(ls_hbm, nlsa_hbm, nlsb_hbm)


ROWS_BLK = 1000


def _head_body(ma_ref, mb_ref, lsa_ref, lsb_ref, noise_ref,
               wd1_ref, bd1_ref, wdx_ref, bdx_ref,
               wc1_ref, bc1_ref, wca_ref, bca_ref,
               x_ref, a_ref, mean_ref, ls_ref):
    mean = ma_ref[...] + mb_ref[...]
    ls = lsa_ref[...] + lsb_ref[...]
    mean_ref[...] = mean
    ls_ref[...] = ls
    noise = noise_ref[...]
    z = noise * jnp.exp(ls) + mean
    h = jnp.maximum(jnp.dot(z, wd1_ref[...], preferred_element_type=jnp.float32)
                    + bd1_ref[...], 0.0)
    x_ref[...] = jnp.maximum(
        jnp.dot(h, wdx_ref[...], preferred_element_type=jnp.float32) + bdx_ref[...], 0.0)
    c = jnp.maximum(jnp.dot(z, wc1_ref[...], preferred_element_type=jnp.float32)
                    + bc1_ref[...], 0.0)
    logits = jnp.dot(c, wca_ref[...], preferred_element_type=jnp.float32) + bca_ref[...]
    col = jax.lax.broadcasted_iota(jnp.int32, logits.shape, 1)
    valid = col < CAT
    logits = jnp.where(valid, logits, -jnp.inf)
    m = jnp.max(logits, axis=-1, keepdims=True)
    e = jnp.where(valid, jnp.exp(logits - m), 0.0)
    a_ref[...] = e / jnp.sum(e, axis=-1, keepdims=True)


def _head(ma, mb, lsa, lsb, noise, Wd1, bd1, WdX, bdX, Wc1, bc1, WcA_p, bcA_p):
    grid = NPR // ROWS_BLK
    row_spec = pl.BlockSpec((ROWS_BLK, H2), lambda i: (i, 0))
    full = lambda a: pl.BlockSpec(a.shape, lambda i: tuple(0 for _ in a.shape))
    args = (ma, mb, lsa, lsb, noise, Wd1, bd1, WdX, bdX, Wc1, bc1, WcA_p, bcA_p)
    out_x, out_a, out_mean, out_ls = pl.pallas_call(
        _head_body,
        grid=(grid,),
        in_specs=[row_spec] * 5 + [full(a) for a in args[5:]],
        out_specs=[pl.BlockSpec((ROWS_BLK, OUT), lambda i: (i, 0)),
                   pl.BlockSpec((ROWS_BLK, 128), lambda i: (i, 0)),
                   row_spec, row_spec],
        out_shape=[jax.ShapeDtypeStruct((NPR, OUT), jnp.float32),
                   jax.ShapeDtypeStruct((NPR, 128), jnp.float32),
                   jax.ShapeDtypeStruct((NPR, H2), jnp.float32),
                   jax.ShapeDtypeStruct((NPR, H2), jnp.float32)],
    )(*args)
    return out_x, out_a, out_mean, out_ls


def kernel(x, edge_index, pos_edge_index, neg_edge_index, temp, W0, b0, W1, b1,
           Wm, bm, Wls, bls, Wd1, bd1, WdX, bdX, Wc1, bc1, WcA, bcA):
    f32 = jnp.float32
    epad = jnp.full((E_PAD - E,), NACC - 1, jnp.int32)
    src_flat = jnp.concatenate([edge_index[0], epad])
    dst_flat = jnp.concatenate([edge_index[1], epad])

    def _with_pad_rows(flat, width, nchunks, npad):
        a = flat.reshape(32, nchunks, width)
        pr = jnp.zeros((32, npad, width), jnp.int32)
        return jnp.concatenate([a, pr], axis=1).reshape(-1, width)

    srcP = _with_pad_rows(src_flat, CW, CHW, 8)
    dstP = dst_flat.reshape(-1, CW)
    deg_cat = jnp.concatenate([src_flat.reshape(-1, 128),
                               dst_flat.reshape(-1, 128)])
    ppad = jnp.zeros((CHP * 128 * 32 - NPR,), jnp.int32)
    pa = _with_pad_rows(jnp.concatenate([pos_edge_index[0], ppad]), 128, CHP, 5)
    pb = _with_pad_rows(jnp.concatenate([pos_edge_index[1], ppad]), 128, CHP, 5)
    na = _with_pad_rows(jnp.concatenate([neg_edge_index[0], ppad]), 128, CHP, 5)
    nb = _with_pad_rows(jnp.concatenate([neg_edge_index[1], ppad]), 128, CHP, 5)
    xp = jnp.zeros((NACC, D_IN), f32).at[:N].set(x)
    zeros128 = jnp.zeros((632, 128), f32)
    ones128 = jnp.ones((128, 128), f32)


    cnt = _deg_kernel(deg_cat, zeros128, ones128)
    ns = lax.rsqrt(jnp.clip(cnt[:NACC, 0], 1.0))
    nd = lax.rsqrt(jnp.clip(cnt[NACC:, 0], 1.0))

    v1 = xp * ns[:, None]
    aggp = _agg1_kernel(srcP, dstP, v1, zeros128)
    agg1 = aggp[:NACC] + aggp[NACC:]
    h1 = jax.nn.relu((agg1 * nd[:, None]) @ W0 + b0)

    t = (h1 * ns[:, None]) @ W1
    a2h0 = _agg1_kernel(srcP, dstP, t[:, :128], zeros128)
    a2h1 = _agg1_kernel(srcP, dstP, t[:, 128:], zeros128)
    agg2 = jnp.concatenate([a2h0[:NACC] + a2h0[NACC:],
                            a2h1[:NACC] + a2h1[NACC:]], axis=1)
    h2 = jax.nn.relu(agg2 * nd[:, None] + b1)

    M = h2 @ Wm + 0.5 * bm
    LS = h2 @ Wls + 0.5 * bls
    (pma, pmb, plsa, plsb,
     nma, nmb, nlsa, nlsb) = _pairs_kernel(M, LS, pa, pb, na, nb)

    kp, kn = jax.random.split(jax.random.key(42))
    noise_p = jax.random.normal(kp, (NPR, H2), f32)
    noise_n = jax.random.normal(kn, (NPR, H2), f32)

    WcA_p = jnp.zeros((H2 // 2, 128), f32).at[:, :CAT].set(WcA)
    bcA_p = jnp.zeros((1, 128), f32).at[0, :CAT].set(bcA)
    head = functools.partial(_head, Wd1=Wd1, bd1=bd1.reshape(1, -1), WdX=WdX,
                             bdX=bdX.reshape(1, -1), Wc1=Wc1,
                             bc1=bc1.reshape(1, -1), WcA_p=WcA_p, bcA_p=bcA_p)
    posX, posA, pos_mean, pos_ls = head(pma, pmb, plsa, plsb, noise_p)
    negX, negA, neg_mean, neg_ls = head(nma, nmb, nlsa, nlsb, noise_n)
    return (posA[:, :CAT], negA[:, :CAT], posX, negX,
            pos_mean, neg_mean, pos_ls, neg_ls)
